# baseline (speedup 1.0000x reference)
import functools

import jax
import jax.numpy as jnp
import numpy as np
from jax import lax
from jax.experimental import pallas as pl
from jax.experimental.pallas import tpu as pltpu
from jax.experimental.pallas import tpu_sc as plsc

C, L, M = 32, 4, 2048
BRANCH, DEPTH = 4, 5
COUNTS = [BRANCH ** d for d in range(DEPTH + 1)]
STARTS = list(np.cumsum([0] + COUNTS))
N = int(STARTS[-1])
NPAD = 1536


@functools.lru_cache(maxsize=None)
def _sc_gather_kernel():
    info = plsc.get_sparse_core_info()
    nw = info.num_cores * info.num_subcores
    b_per_w = NPAD // nw
    mesh = plsc.VectorSubcoreMesh(core_axis_name="c", subcore_axis_name="s")

    @functools.partial(
        pl.kernel, mesh=mesh,
        out_type=jax.ShapeDtypeStruct((NPAD, 128), jnp.float32),
        scratch_types=[
            pltpu.VMEM((b_per_w,), jnp.int32),
            pltpu.VMEM((b_per_w, 128), jnp.float32),
            pltpu.SemaphoreType.DMA,
        ],
    )
    def gather(table_hbm, idx_hbm, out_hbm, idx_v, rows_v, sem):
        wid = lax.axis_index("s") * info.num_cores + lax.axis_index("c")
        base = wid * b_per_w
        pltpu.sync_copy(idx_hbm.at[pl.ds(base, b_per_w)], idx_v)
        pltpu.async_copy(table_hbm.at[idx_v], rows_v, sem).wait()
        pltpu.sync_copy(rows_v, out_hbm.at[pl.ds(base, b_per_w)])

    return gather

_HP = lax.Precision.HIGHEST


def _mm(a, b):
    return lax.dot_general(a, b, (((1,), (0,)), ((), ())), precision=_HP,
                           preferred_element_type=jnp.float32)


def _mask(shape, fn):
    r = lax.broadcasted_iota(jnp.int32, shape, 0)
    c = lax.broadcasted_iota(jnp.int32, shape, 1)
    return fn(r, c).astype(jnp.float32)


def _htmm_kernel(bt_ref, a_up_ref, a_jk_ref, pit_ref, spr_ref, spc_ref,
                 bg_ref, out_ref):
    BT = bt_ref[...]
    Bg = bg_ref[:, :C]
    A_up = a_up_ref[...]
    A_jk = a_jk_ref[...]
    PiT = pit_ref[...]
    SPr = spr_ref[...]
    SPc = spc_ref[...]

    m_up = jnp.max(A_up, axis=1, keepdims=True)
    e_up = jnp.exp(A_up - m_up)
    sm_A_up = e_up / jnp.sum(e_up, axis=1, keepdims=True)

    m_jk = jnp.max(A_jk, axis=0, keepdims=True)
    z_jk = A_jk - m_jk
    e_jk = jnp.exp(z_jk)
    s_jk = jnp.sum(e_jk, axis=0, keepdims=True)
    sm_A_jk = e_jk / s_jk
    log_sm_A_jk = z_jk - jnp.log(s_jk)

    m_pi = jnp.max(PiT, axis=1, keepdims=True)
    z_pi = PiT - m_pi
    e_pi = jnp.exp(z_pi)
    s_pi = jnp.sum(e_pi, axis=1, keepdims=True)
    sm_PiT = e_pi / s_pi
    log_sm_PiT = z_pi - jnp.log(s_pi)

    m_sp = jnp.max(SPr, axis=1, keepdims=True)
    z_sp = SPr - m_sp
    e_sp = jnp.exp(z_sp)
    s_sp = jnp.sum(e_sp, axis=1, keepdims=True)
    sm_SPr = e_sp / s_sp
    log_sm_SPr = z_sp - jnp.log(s_sp)
    del sm_SPr

    m_spc = jnp.max(SPc, axis=0, keepdims=True)
    e_spc = jnp.exp(SPc - m_spc)
    sm_SPc = e_spc / jnp.sum(e_spc, axis=0, keepdims=True)

    m_b = jnp.max(BT, axis=0, keepdims=True)
    lse = m_b + jnp.log(jnp.sum(jnp.exp(BT - m_b), axis=0, keepdims=True))

    nL = COUNTS[DEPTH]
    E2T = _mask((nL, L), lambda r, c: (r % BRANCH) == c)
    pi_lv = _mm(E2T, sm_PiT)
    b_lv = jnp.exp(BT[STARTS[DEPTH]:STARTS[DEPTH + 1], :] - lse)
    bl = pi_lv * b_lv
    denom = jnp.sum(bl, axis=0, keepdims=True)
    betas = [None] * (DEPTH + 1)
    tbetas = [None] * DEPTH
    betas[DEPTH] = bl / denom

    for d in range(DEPTH - 1, -1, -1):
        n_par, n_ch = COUNTS[d], COUNTS[d + 1]
        b_ch = betas[d + 1]
        Mk4 = _mask((n_ch, L), lambda r, c: (r % BRANCH) == c)
        sp_col = _mm(Mk4, sm_SPc)
        t_child = jnp.zeros((n_ch, C), jnp.float32)
        for k in range(BRANCH):
            Ak = sm_A_up[k * C:(k + 1) * C, :]
            t_child = t_child + Mk4[:, k:k + 1] * _mm(b_ch, Ak)
        t_child = sp_col * t_child
        G = _mask((n_par, n_ch), lambda r, c: (c // BRANCH) == r)
        t_beta = _mm(G, t_child)
        tbetas[d] = t_beta
        smB = jnp.exp(Bg[STARTS[d]:STARTS[d + 1], :] - lse)
        bu = t_beta * smB
        betas[d] = bu / jnp.sum(bu, axis=1, keepdims=True)

    E_mod4 = _mask((L, C * L), lambda r, c: (c % BRANCH) == r)
    logSP_jk = _mm(log_sm_SPr, E_mod4)
    G2 = sm_A_jk * (log_sm_A_jk + logSP_jk)

    eps = [None] * (DEPTH + 1)
    eps[0] = betas[0]
    ll_asp = jnp.float32(0.0)
    for d in range(DEPTH):
        n_par, n_ch = COUNTS[d], COUNTS[d + 1]
        R = eps[d] / tbetas[d]
        S = _mm(R, sm_A_jk)
        T = _mm(R, G2)
        Expand = _mask((n_ch, n_par), lambda r, c: (r // BRANCH) == c)
        Mk4 = _mask((n_ch, L), lambda r, c: (r % BRANCH) == c)
        sp_col = _mm(Mk4, sm_SPc)
        Ssel = jnp.zeros((n_ch, C), jnp.float32)
        Tsel = jnp.zeros((n_ch, C), jnp.float32)
        for k in range(BRANCH):
            Qk = _mask((C * L, C), lambda r, c: r == (c * BRANCH + k))
            Ssel = Ssel + Mk4[:, k:k + 1] * _mm(Expand, _mm(S, Qk))
            Tsel = Tsel + Mk4[:, k:k + 1] * _mm(Expand, _mm(T, Qk))
        bsp = betas[d + 1] * sp_col
        eps[d + 1] = bsp * Ssel
        ll_asp = ll_asp + jnp.sum(bsp * Tsel)

    b_lhood = jnp.float32(0.0)
    for d in range(DEPTH + 1):
        b_lhood = b_lhood + jnp.sum(
            eps[d] * (Bg[STARTS[d]:STARTS[d + 1], :] - lse))
    log_pi_lv = _mm(E2T, log_sm_PiT)
    pi_lhood = jnp.sum(eps[DEPTH] * log_pi_lv)

    out_ref[...] = jnp.reshape(ll_asp + b_lhood + pi_lhood, (1, 1))


def kernel(A, B, Pi, SP, labels, pos, leaves, levels):
    del pos, leaves, levels
    A = A.astype(jnp.float32)
    BT = jnp.transpose(B.astype(jnp.float32))
    A_up = jnp.transpose(A, (2, 1, 0)).reshape(L * C, C)
    A_jk = A.reshape(C, C * L)
    PiT = jnp.transpose(Pi.astype(jnp.float32))
    SPr = SP.astype(jnp.float32).reshape(1, L)
    SPc = SP.astype(jnp.float32).reshape(L, 1)
    lbl = jnp.concatenate([jnp.asarray(labels, jnp.int32),
                           jnp.zeros((NPAD - N,), jnp.int32)])
    BT128 = jnp.pad(BT, ((0, 0), (0, 128 - C)))
    Bg = _sc_gather_kernel()(BT128, lbl)
    out = pl.pallas_call(
        _htmm_kernel,
        out_shape=jax.ShapeDtypeStruct((1, 1), jnp.float32),
    )(BT, A_up, A_jk, PiT, SPr, SPc, Bg)
    return out[0, 0]


# speedup vs baseline: 1.3654x; 1.3654x over previous
import functools

import jax
import jax.numpy as jnp
import numpy as np
from jax import lax
from jax.experimental import pallas as pl
from jax.experimental.pallas import tpu as pltpu
from jax.experimental.pallas import tpu_sc as plsc

C, L, M = 32, 4, 2048
BRANCH, DEPTH = 4, 5
COUNTS = [BRANCH ** d for d in range(DEPTH + 1)]
STARTS = list(np.cumsum([0] + COUNTS))
N = int(STARTS[-1])
NPAD = 1536
CL = C * L


@functools.lru_cache(maxsize=None)
def _sc_gather_kernel():
    info = plsc.get_sparse_core_info()
    nw = info.num_cores * info.num_subcores
    b_per_w = NPAD // nw
    mesh = plsc.VectorSubcoreMesh(core_axis_name="c", subcore_axis_name="s")

    @functools.partial(
        pl.kernel, mesh=mesh,
        out_type=jax.ShapeDtypeStruct((NPAD, 128), jnp.float32),
        scratch_types=[
            pltpu.VMEM((b_per_w,), jnp.int32),
            pltpu.VMEM((b_per_w, 128), jnp.float32),
            pltpu.SemaphoreType.DMA,
        ],
    )
    def gather(table_hbm, idx_hbm, out_hbm, idx_v, rows_v, sem):
        wid = lax.axis_index("s") * info.num_cores + lax.axis_index("c")
        base = wid * b_per_w
        pltpu.sync_copy(idx_hbm.at[pl.ds(base, b_per_w)], idx_v)
        pltpu.async_copy(table_hbm.at[idx_v], rows_v, sem).wait()
        pltpu.sync_copy(rows_v, out_hbm.at[pl.ds(base, b_per_w)])

    return gather


_HP = lax.Precision.HIGHEST


def _mm(a, b):
    return lax.dot_general(a, b, (((1,), (0,)), ((), ())), precision=_HP,
                           preferred_element_type=jnp.float32)


def _mask(shape, fn):
    r = lax.broadcasted_iota(jnp.int32, shape, 0)
    c = lax.broadcasted_iota(jnp.int32, shape, 1)
    return fn(r, c).astype(jnp.float32)


def _htmm_kernel(bt_ref, a_kj_ref, pit_ref, spr_ref, spc_ref,
                 bg_ref, out_ref):
    BT = bt_ref[:, :C]
    Bg = bg_ref[:, :C]
    A_kj = a_kj_ref[...]
    PiT = pit_ref[...]
    SPr = spr_ref[...]
    SPc = spc_ref[...]

    m_kj = jnp.max(A_kj, axis=0, keepdims=True)
    z_kj = A_kj - m_kj
    e_kj = jnp.exp(z_kj)
    s_kj = jnp.sum(e_kj, axis=0, keepdims=True)
    sm_A_kj = e_kj / s_kj
    log_sm_A_kj = z_kj - jnp.log(s_kj)

    m_pi = jnp.max(PiT, axis=1, keepdims=True)
    z_pi = PiT - m_pi
    e_pi = jnp.exp(z_pi)
    s_pi = jnp.sum(e_pi, axis=1, keepdims=True)
    sm_PiT = e_pi / s_pi
    log_sm_PiT = z_pi - jnp.log(s_pi)

    m_sp = jnp.max(SPr, axis=1, keepdims=True)
    z_sp = SPr - m_sp
    e_sp = jnp.exp(z_sp)
    s_sp = jnp.sum(e_sp, axis=1, keepdims=True)
    sm_SPr = e_sp / s_sp
    log_sm_SPr = z_sp - jnp.log(s_sp)

    m_spc = jnp.max(SPc, axis=0, keepdims=True)
    e_spc = jnp.exp(SPc - m_spc)
    sm_SPc = e_spc / jnp.sum(e_spc, axis=0, keepdims=True)

    E_k128 = _mask((L, CL), lambda r, c: (c // C) == r)
    sp_rows = _mm(jnp.concatenate([sm_SPr, log_sm_SPr], axis=0), E_k128)
    sp_row = sp_rows[0:1, :]
    log_sp_row = sp_rows[1:2, :]

    m_b = jnp.max(BT, axis=0, keepdims=True)
    lse = m_b + jnp.log(jnp.sum(jnp.exp(BT - m_b), axis=0, keepdims=True))

    nL = COUNTS[DEPTH]
    E2T = _mask((nL, L), lambda r, c: (r % BRANCH) == c)
    pi_both = _mm(E2T, jnp.concatenate([sm_PiT, log_sm_PiT], axis=1))
    pi_lv = pi_both[:, :C]
    log_pi_lv = pi_both[:, C:]
    b_lv = jnp.exp(BT[STARTS[DEPTH]:STARTS[DEPTH + 1], :] - lse)
    bl = pi_lv * b_lv
    denom = jnp.sum(bl, axis=0, keepdims=True)
    betas = [None] * (DEPTH + 1)
    tbetas = [None] * DEPTH
    bsps = [None] * (DEPTH + 1)
    betas[DEPTH] = bl / denom

    lvl = {}
    for n_ch in (BRANCH ** e for e in range(1, DEPTH + 1)):
        n_par = n_ch // BRANCH
        Mk4 = _mask((n_ch, L), lambda r, c: (r % BRANCH) == c)
        sp_col = _mm(Mk4, sm_SPc)
        G = _mask((n_par, n_ch), lambda r, c: (c // BRANCH) == r)
        Expand = _mask((n_ch, n_par), lambda r, c: (r // BRANCH) == c)
        lvl[n_ch] = (Mk4, sp_col, G, Expand)

    def _pick(X, Mk4, base):
        acc = Mk4[:, 0:1] * X[:, base:base + C]
        for k in range(1, BRANCH):
            acc = acc + Mk4[:, k:k + 1] * X[:, base + k * C:base + (k + 1) * C]
        return acc

    A_all = jnp.concatenate(
        [jnp.transpose(sm_A_kj[:, k * C:(k + 1) * C]) for k in range(BRANCH)],
        axis=1)

    for d in range(DEPTH - 1, -1, -1):
        n_par, n_ch = COUNTS[d], COUNTS[d + 1]
        Mk4, sp_col, G, _ = lvl[n_ch]
        t_all = _mm(betas[d + 1], A_all)
        t_child = sp_col * _pick(t_all, Mk4, 0)
        t_beta = _mm(G, t_child)
        tbetas[d] = t_beta
        smB = jnp.exp(Bg[STARTS[d]:STARTS[d + 1], :] - lse)
        bu = t_beta * smB
        betas[d] = bu / jnp.sum(bu, axis=1, keepdims=True)

    G2 = sm_A_kj * (log_sm_A_kj + log_sp_row)
    AG = jnp.concatenate([sm_A_kj, G2], axis=1)

    eps = [None] * (DEPTH + 1)
    eps[0] = betas[0]
    ll_asp = jnp.float32(0.0)
    for d in range(DEPTH):
        n_par, n_ch = COUNTS[d], COUNTS[d + 1]
        Mk4, sp_col, _, Expand = lvl[n_ch]
        R = eps[d] / tbetas[d]
        ST = _mm(R, AG)
        XT = _mm(Expand, ST)
        bsp = betas[d + 1] * sp_col
        eps[d + 1] = bsp * _pick(XT, Mk4, 0)
        ll_asp = ll_asp + jnp.sum(bsp * _pick(XT, Mk4, CL))

    b_lhood = jnp.float32(0.0)
    for d in range(DEPTH + 1):
        b_lhood = b_lhood + jnp.sum(
            eps[d] * (Bg[STARTS[d]:STARTS[d + 1], :] - lse))
    pi_lhood = jnp.sum(eps[DEPTH] * log_pi_lv)

    out_ref[...] = jnp.reshape(ll_asp + b_lhood + pi_lhood, (1, 1))


def kernel(A, B, Pi, SP, labels, pos, leaves, levels):
    del pos, leaves, levels
    A = A.astype(jnp.float32)
    BT128 = jnp.pad(jnp.transpose(B.astype(jnp.float32)),
                    ((0, 0), (0, 128 - C)))
    A_kj = jnp.transpose(A, (0, 2, 1)).reshape(C, CL)
    PiT = jnp.transpose(Pi.astype(jnp.float32))
    SPr = SP.astype(jnp.float32).reshape(1, L)
    SPc = SP.astype(jnp.float32).reshape(L, 1)
    lbl = jnp.concatenate([jnp.asarray(labels, jnp.int32),
                           jnp.zeros((NPAD - N,), jnp.int32)])
    Bg = _sc_gather_kernel()(BT128, lbl)
    out = pl.pallas_call(
        _htmm_kernel,
        out_shape=jax.ShapeDtypeStruct((1, 1), jnp.float32),
    )(BT128, A_kj, PiT, SPr, SPc, Bg)
    return out[0, 0]
